# baseline (speedup 1.0000x reference)
import functools

import numpy as np

import jax
import jax.numpy as jnp
from jax import lax
from jax.experimental import pallas as pl
from jax.experimental.pallas import tpu as pltpu
from jax.experimental.pallas import tpu_sc as plsc

_RATIO = 0.75
_B, _T, _C = 256, 1024, 192
_R = int(_T * (1 - _RATIO))
_NC, _NS = 2, 16
_NW = _NC * _NS
_BPW = _B // _NW
_WIN = 128
_NWPB = _T // _WIN
_NV = _BPW * _NWPB
_NRING = 3
_GMAX = 4
_LANES = 16
_WINW = _WIN * _C
_OUTW = _R * _C


def _build_schedule(fwd_np):
    srcs = np.sort(fwd_np[:, :_R], axis=1)
    order = np.argsort(fwd_np[:, :_R], axis=1)
    sched = np.zeros((_NW, _NV, _GMAX, _LANES), dtype=np.int32)
    for wid in range(_NW):
        for bb in range(_BPW):
            b = wid * _BPW + bb
            s_all, d_all = srcs[b], order[b]
            for w in range(_NWPB):
                m = (s_all // _WIN) == w
                s, d = s_all[m], d_all[m]
                cnt = int(s.size)
                assert 1 <= cnt <= _GMAX * _LANES, (b, w, cnt)
                gcnt = -(-cnt // _LANES)
                pad = gcnt * _LANES - cnt
                s = np.concatenate([s, np.full(pad, s[0])])
                d = np.concatenate([d, np.full(pad, d[0])])
                vv = bb * _NWPB + w
                slot = vv % _NRING
                packed = ((slot * _WINW + (s % _WIN) * _C)
                          | (d << 17) | (gcnt << 29)).astype(np.int32)
                sched[wid, vv, :gcnt] = packed.reshape(gcnt, _LANES)
    return sched


def _body(table, sched_hbm, out, sched_v, stage, outb, ssem, psem):
    wid = lax.axis_index("s") * _NC + lax.axis_index("c")
    base_b = wid * _BPW
    pltpu.sync_copy(sched_hbm.at[wid], sched_v)

    def start_stage(bb, w, slot):
        src = table.at[base_b + bb, pl.ds(w * _WINW, _WINW)]
        return pltpu.async_copy(
            src, stage.at[pl.ds(slot * _WINW, _WINW)], ssem.at[slot])

    for vv in range(_NRING):
        start_stage(vv // _NWPB, vv % _NWPB, vv % _NRING)

    def window(vv, carry):
        bb = vv // _NWPB
        w = lax.rem(vv, _NWPB)
        slot = lax.rem(vv, _NRING)
        pltpu.make_async_copy(
            table.at[0, pl.ds(0, _WINW)],
            stage.at[pl.ds(slot * _WINW, _WINW)], ssem.at[slot]).wait()
        @pl.when(jnp.logical_and(w == 0, vv > 0))
        def _():
            pltpu.make_async_copy(outb, out.at[0], psem).wait()

        v0 = sched_v[vv, 0, :]
        gcnt = jnp.max(lax.shift_right_logical(v0, 29))

        def group(gg, c2):
            vg = sched_v[vv, gg, :]
            s0 = lax.bitwise_and(vg, (1 << 17) - 1)
            d0 = lax.bitwise_and(lax.shift_right_logical(vg, 17), 511) * _C

            def w_body(i, carry2):
                s, d = carry2
                for _ in range(16):
                    x = plsc.load_gather(stage, [s])
                    plsc.store_scatter(outb, [d], x)
                    s = s + 1
                    d = d + 1
                return (s, d)

            lax.fori_loop(0, _C // 16, w_body, (s0, d0), unroll=False)
            return c2

        lax.fori_loop(0, gcnt, group, 0, unroll=False)

        @pl.when(vv + _NRING < _NV)
        def _():
            nxt = vv + _NRING
            start_stage(nxt // _NWPB, lax.rem(nxt, _NWPB), slot)

        @pl.when(w == _NWPB - 1)
        def _():
            pltpu.async_copy(outb, out.at[base_b + bb], psem)
        return carry

    lax.fori_loop(0, _NV, window, 0, unroll=False)
    pltpu.make_async_copy(outb, out.at[0], psem).wait()


_gather = functools.partial(
    pl.kernel,
    out_type=jax.ShapeDtypeStruct((_B, _OUTW), jnp.float32),
    mesh=plsc.VectorSubcoreMesh(core_axis_name="c", subcore_axis_name="s"),
    scratch_types=[
        pltpu.VMEM((_NV, _GMAX, _LANES), jnp.int32),
        pltpu.VMEM((_NRING * _WINW,), jnp.float32),
        pltpu.VMEM((_OUTW,), jnp.float32),
        pltpu.SemaphoreType.DMA((_NRING,)),
        pltpu.SemaphoreType.DMA,
    ],
    compiler_params=pltpu.CompilerParams(
        use_tc_tiling_on_sc=False, needs_layout_passes=False),
)(_body)


def _perm_constants():
    import contextlib
    try:
        cpu_mesh = jax.sharding.Mesh(np.array(jax.devices("cpu")[:1]), ("_",))
        ctx = jax.set_mesh(cpu_mesh)
    except Exception:
        ctx = contextlib.nullcontext()
    with ctx:
        keys = jax.random.split(jax.random.key(42), _B)
        fwd = jax.vmap(lambda k: jax.random.permutation(k, _T))(keys).astype(
            jnp.int64)
        bwd = jnp.argsort(fwd, axis=1)
    return np.asarray(fwd), np.asarray(bwd)


_FWD_NP, _BWD_NP = _perm_constants()
_CACHE = []


def kernel(patches):
    if not _CACHE:
        _CACHE.append(jnp.asarray(_build_schedule(_FWD_NP)))
    sched = _CACHE[0]
    out = _gather(patches.reshape(_B, _T * _C), sched)
    return (out.reshape(_B, _R, _C),
            jnp.asarray(_FWD_NP), jnp.asarray(_BWD_NP))


# speedup vs baseline: 1.1654x; 1.1654x over previous
import functools

import numpy as np

import jax
import jax.numpy as jnp
from jax import lax
from jax.experimental import pallas as pl
from jax.experimental.pallas import tpu as pltpu
from jax.experimental.pallas import tpu_sc as plsc

_RATIO = 0.75
_B, _T, _C = 256, 1024, 192
_R = int(_T * (1 - _RATIO))
_NC, _NS = 2, 16
_NW = _NC * _NS
_BPW = _B // _NW
_WIN = 128
_NWPB = _T // _WIN
_NV = _BPW * _NWPB
_NRING = 3
_GMAX = 4
_LANES = 16
_STEP = 16


def _build_schedule(fwd_np):
    srcs = np.sort(fwd_np[:, :_R], axis=1)
    order = np.argsort(fwd_np[:, :_R], axis=1)
    sched = np.zeros((_NW, _NV, _GMAX, _LANES), dtype=np.int32)
    for wid in range(_NW):
        for bb in range(_BPW):
            b = wid * _BPW + bb
            s_all, d_all = srcs[b], order[b]
            for w in range(_NWPB):
                m = (s_all // _WIN) == w
                s, d = s_all[m], d_all[m]
                cnt = int(s.size)
                assert 1 <= cnt <= _GMAX * _LANES, (b, w, cnt)
                gcnt = -(-cnt // _LANES)
                pad = gcnt * _LANES - cnt
                s = np.concatenate([s, np.full(pad, s[0])])
                d = np.concatenate([d, np.full(pad, d[0])])
                vv = bb * _NWPB + w
                slot = vv % _NRING
                ring_row = slot * _WIN + (s % _WIN)
                packed = (ring_row | (d << 9) | (gcnt << 29)).astype(np.int32)
                sched[wid, vv, :gcnt] = packed.reshape(gcnt, _LANES)
    return sched


def _body(table, sched_hbm, out, sched_v, stage, outb, ssem, psem):
    wid = lax.axis_index("s") * _NC + lax.axis_index("c")
    base_b = wid * _BPW
    pltpu.sync_copy(sched_hbm.at[wid], sched_v)

    def start_stage(bb, w, slot):
        src = table.at[base_b + bb, pl.ds(w * _WIN, _WIN), :]
        return pltpu.async_copy(
            src, stage.at[pl.ds(slot * _WIN, _WIN), :], ssem.at[slot])

    for vv in range(_NRING):
        start_stage(vv // _NWPB, vv % _NWPB, vv % _NRING)

    def window(vv, carry):
        bb = vv // _NWPB
        w = lax.rem(vv, _NWPB)
        slot = lax.rem(vv, _NRING)
        pltpu.make_async_copy(
            table.at[0, pl.ds(0, _WIN), :],
            stage.at[pl.ds(slot * _WIN, _WIN), :], ssem.at[slot]).wait()
        @pl.when(jnp.logical_and(w == 0, vv > 0))
        def _():
            pltpu.make_async_copy(outb, out.at[0], psem).wait()

        v0 = sched_v[vv, 0, :]
        gcnt = jnp.max(lax.shift_right_logical(v0, 29))

        def group(gg, c2):
            vg = sched_v[vv, gg, :]
            row = lax.bitwise_and(vg, 511)
            dst = lax.bitwise_and(lax.shift_right_logical(vg, 9), 255)

            def w_body(i, col0):
                cols = [col0]
                for _ in range(_STEP - 1):
                    cols.append(cols[-1] + 1)
                xs = [plsc.load_gather(stage, [row, c]) for c in cols]
                for c, x in zip(cols, xs):
                    plsc.store_scatter(outb, [dst, c], x)
                return cols[-1] + 1

            lax.fori_loop(0, _C // _STEP, w_body,
                          jnp.zeros((_LANES,), jnp.int32), unroll=False)
            return c2

        lax.fori_loop(0, gcnt, group, 0, unroll=False)

        @pl.when(vv + _NRING < _NV)
        def _():
            nxt = vv + _NRING
            start_stage(nxt // _NWPB, lax.rem(nxt, _NWPB), slot)

        @pl.when(w == _NWPB - 1)
        def _():
            pltpu.async_copy(outb, out.at[base_b + bb], psem)
        return carry

    lax.fori_loop(0, _NV, window, 0, unroll=False)
    pltpu.make_async_copy(outb, out.at[0], psem).wait()


_gather = functools.partial(
    pl.kernel,
    out_type=jax.ShapeDtypeStruct((_B, _R, _C), jnp.float32),
    mesh=plsc.VectorSubcoreMesh(core_axis_name="c", subcore_axis_name="s"),
    scratch_types=[
        pltpu.VMEM((_NV, _GMAX, _LANES), jnp.int32),
        pltpu.VMEM((_NRING * _WIN, _C), jnp.float32),
        pltpu.VMEM((_R, _C), jnp.float32),
        pltpu.SemaphoreType.DMA((_NRING,)),
        pltpu.SemaphoreType.DMA,
    ],
    compiler_params=pltpu.CompilerParams(
        use_tc_tiling_on_sc=False, needs_layout_passes=False),
)(_body)


def _perm_constants():
    import contextlib
    try:
        cpu_mesh = jax.sharding.Mesh(np.array(jax.devices("cpu")[:1]), ("_",))
        ctx = jax.set_mesh(cpu_mesh)
    except Exception:
        ctx = contextlib.nullcontext()
    with ctx:
        keys = jax.random.split(jax.random.key(42), _B)
        fwd = jax.vmap(lambda k: jax.random.permutation(k, _T))(keys).astype(
            jnp.int64)
        bwd = jnp.argsort(fwd, axis=1)
    return np.asarray(fwd), np.asarray(bwd)


_FWD_NP, _BWD_NP = _perm_constants()
_CACHE = []


def kernel(patches):
    if not _CACHE:
        _CACHE.append(jnp.asarray(_build_schedule(_FWD_NP)))
    sched = _CACHE[0]
    out = _gather(patches, sched)
    return (out, jnp.asarray(_FWD_NP), jnp.asarray(_BWD_NP))


# speedup vs baseline: 1.4550x; 1.2485x over previous
import functools

import numpy as np

import jax
import jax.numpy as jnp
from jax import lax
from jax.experimental import pallas as pl
from jax.experimental.pallas import tpu as pltpu
from jax.experimental.pallas import tpu_sc as plsc

_RATIO = 0.75
_B, _T, _C = 256, 1024, 192
_R = int(_T * (1 - _RATIO))
_NC, _NS = 2, 16
_NW = _NC * _NS
_BPW = _B // _NW
_WIN = 64
_NWPB = _T // _WIN
_NV = _BPW * _NWPB
_NRING = 3
_GMAX = 2
_LANES = 16
_SCHED_ROWS = _NV * _GMAX * _LANES // 128


def _build_schedule(fwd_np):
    srcs = np.sort(fwd_np[:, :_R], axis=1)
    order = np.argsort(fwd_np[:, :_R], axis=1)
    sched = np.zeros((_NW, _NV, _GMAX, _LANES), dtype=np.int32)
    for wid in range(_NW):
        for bb in range(_BPW):
            b = wid * _BPW + bb
            s_all, d_all = srcs[b], order[b]
            for w in range(_NWPB):
                m = (s_all // _WIN) == w
                s, d = s_all[m], d_all[m]
                cnt = int(s.size)
                assert 1 <= cnt <= _GMAX * _LANES, (b, w, cnt)
                gcnt = -(-cnt // _LANES)
                pad = gcnt * _LANES - cnt
                s = np.concatenate([s, np.full(pad, s[0])])
                d = np.concatenate([d, np.full(pad, d[0])])
                vv = bb * _NWPB + w
                packed = ((s % _WIN) | (d << 6) | (gcnt << 29)).astype(
                    np.int32)
                sched[wid, vv, :gcnt] = packed.reshape(gcnt, _LANES)
    return sched.reshape(_NW, _SCHED_ROWS, 128)


def _body(table, sched_hbm, out, sched_v, stage, outb, ssem, psem):
    wid = lax.axis_index("s") * _NC + lax.axis_index("c")
    base_b = wid * _BPW
    pltpu.sync_copy(sched_hbm.at[wid], sched_v.at[0])

    def start_stage(bb, w, slot):
        src = table.at[base_b + bb, pl.ds(w * _WIN, _WIN), :]
        return pltpu.async_copy(src, stage.at[slot], ssem.at[slot])

    for vv in range(_NRING):
        start_stage(vv // _NWPB, vv % _NWPB, vv % _NRING)

    def wait_stage(slot):
        pltpu.make_async_copy(
            table.at[0, pl.ds(0, _WIN), :], stage.at[slot],
            ssem.at[slot]).wait()

    def wait_put():
        pltpu.make_async_copy(outb.at[0], out.at[0], psem).wait()

    zero16 = jnp.zeros((_LANES,), jnp.int32)

    def window(vv, carry):
        bb = vv // _NWPB
        w = lax.rem(vv, _NWPB)
        slot = lax.rem(vv, _NRING)
        slot_v = zero16 + slot
        wait_stage(slot)
        @pl.when(jnp.logical_and(w == 0, vv > 0))
        def _():
            wait_put()

        e0 = vv * _GMAX * _LANES
        r0 = lax.div(e0, 128)
        c0 = lax.rem(e0, 128)
        v0 = sched_v[0, r0, pl.ds(c0, _LANES)]
        gcnt = jnp.max(lax.shift_right_logical(v0, 29))

        def group(gg, c2):
            e = e0 + gg * _LANES
            vg = sched_v[0, lax.div(e, 128), pl.ds(lax.rem(e, 128), _LANES)]
            row = lax.bitwise_and(vg, 63)
            dst = lax.bitwise_and(lax.shift_right_logical(vg, 6), 255)

            def step(i, col0):
                cols = [col0]
                for _ in range(_LANES - 1):
                    cols.append(cols[-1] + 1)
                xs = [plsc.load_gather(stage, [slot_v, row, c])
                      for c in cols]
                for c, x in zip(cols, xs):
                    plsc.store_scatter(outb, [zero16, dst, c], x)
                return cols[-1] + 1

            lax.fori_loop(0, _C // _LANES, step, zero16, unroll=False)
            return c2

        lax.fori_loop(0, gcnt, group, 0, unroll=False)

        @pl.when(vv + _NRING < _NV)
        def _():
            nxt = vv + _NRING
            start_stage(nxt // _NWPB, lax.rem(nxt, _NWPB), slot)

        @pl.when(w == _NWPB - 1)
        def _():
            pltpu.async_copy(outb.at[0], out.at[base_b + bb], psem)
        return carry

    lax.fori_loop(0, _NV, window, 0, unroll=False)
    wait_put()


_gather = functools.partial(
    pl.kernel,
    out_type=jax.ShapeDtypeStruct((_B, _R, _C), jnp.float32),
    mesh=plsc.VectorSubcoreMesh(core_axis_name="c", subcore_axis_name="s"),
    scratch_types=[
        pltpu.VMEM((1, _SCHED_ROWS, 128), jnp.int32),
        pltpu.VMEM((_NRING, _WIN, _C), jnp.float32),
        pltpu.VMEM((1, _R, _C), jnp.float32),
        pltpu.SemaphoreType.DMA((_NRING,)),
        pltpu.SemaphoreType.DMA,
    ],
    compiler_params=pltpu.CompilerParams(needs_layout_passes=False),
)(_body)


def _perm_constants():
    import contextlib
    try:
        cpu_mesh = jax.sharding.Mesh(np.array(jax.devices("cpu")[:1]), ("_",))
        ctx = jax.set_mesh(cpu_mesh)
    except Exception:
        ctx = contextlib.nullcontext()
    with ctx:
        keys = jax.random.split(jax.random.key(42), _B)
        fwd = jax.vmap(lambda k: jax.random.permutation(k, _T))(keys).astype(
            jnp.int64)
        bwd = jnp.argsort(fwd, axis=1)
    return np.asarray(fwd), np.asarray(bwd)


_FWD_NP, _BWD_NP = _perm_constants()
_CACHE = []


def kernel(patches):
    if not _CACHE:
        _CACHE.append(jnp.asarray(_build_schedule(_FWD_NP)))
    sched = _CACHE[0]
    out = _gather(patches, sched)
    return (out, jnp.asarray(_FWD_NP), jnp.asarray(_BWD_NP))


# speedup vs baseline: 8.5652x; 5.8868x over previous
import functools

import numpy as np

import jax
import jax.numpy as jnp
from jax import lax
from jax.experimental import pallas as pl
from jax.experimental.pallas import tpu as pltpu
from jax.experimental.pallas import tpu_sc as plsc

_RATIO = 0.75
_B, _T, _C = 256, 1024, 192
_R = int(_T * (1 - _RATIO))
_NC, _NS = 2, 16
_NW = _NC * _NS
_BPW = _B // _NW
_WIN = 128
_NWPB = _T // _WIN
_NV = _BPW * _NWPB
_NRING = 3
_GMAX = 4
_LANES = 16
_SCHED_ROWS = _NV * _GMAX * _LANES // 128


def _build_schedule(fwd_np):
    srcs = np.sort(fwd_np[:, :_R], axis=1)
    order = np.argsort(fwd_np[:, :_R], axis=1)
    sched = np.zeros((_NW, _NV, _GMAX, _LANES), dtype=np.int32)
    for wid in range(_NW):
        for bb in range(_BPW):
            b = wid * _BPW + bb
            s_all, d_all = srcs[b], order[b]
            for w in range(_NWPB):
                m = (s_all // _WIN) == w
                s, d = s_all[m], d_all[m]
                cnt = int(s.size)
                assert 1 <= cnt <= _GMAX * _LANES, (b, w, cnt)
                gcnt = -(-cnt // _LANES)
                pad = gcnt * _LANES - cnt
                s = np.concatenate([s, np.full(pad, s[0])])
                d = np.concatenate([d, np.full(pad, d[0])])
                vv = bb * _NWPB + w
                packed = ((s % _WIN) | (d << 7) | (gcnt << 29)).astype(
                    np.int32)
                sched[wid, vv, :gcnt] = packed.reshape(gcnt, _LANES)
    return sched.reshape(_NW, _SCHED_ROWS, 128)


def _body(table, sched_hbm, out, sched_v, stage, outb, ssem, psem):
    wid = lax.axis_index("s") * _NC + lax.axis_index("c")
    base_b = wid * _BPW
    pltpu.sync_copy(sched_hbm.at[wid], sched_v.at[0])

    def start_stage(bb, w, slot):
        src = table.at[base_b + bb, :, pl.ds(w * _WIN, _WIN)]
        return pltpu.async_copy(src, stage.at[slot], ssem.at[slot])

    for vv in range(_NRING):
        start_stage(vv // _NWPB, vv % _NWPB, vv % _NRING)

    def wait_stage(slot):
        pltpu.make_async_copy(
            table.at[0, :, pl.ds(0, _WIN)], stage.at[slot],
            ssem.at[slot]).wait()

    def wait_put():
        pltpu.make_async_copy(outb.at[0], out.at[0], psem).wait()

    zero16 = jnp.zeros((_LANES,), jnp.int32)

    def window(vv, carry):
        bb = vv // _NWPB
        w = lax.rem(vv, _NWPB)
        slot = lax.rem(vv, _NRING)
        slot_v = zero16 + slot
        wait_stage(slot)
        @pl.when(jnp.logical_and(w == 0, vv > 0))
        def _():
            wait_put()

        e0 = vv * _GMAX * _LANES
        r0 = lax.div(e0, 128)
        c0 = lax.rem(e0, 128)
        v0 = sched_v[0, r0, pl.ds(c0, _LANES)]
        gcnt = jnp.max(lax.shift_right_logical(v0, 29))

        def group(gg, c2):
            e = e0 + gg * _LANES
            vg = sched_v[0, lax.div(e, 128), pl.ds(lax.rem(e, 128), _LANES)]
            src = lax.bitwise_and(vg, 127)
            dst = lax.bitwise_and(lax.shift_right_logical(vg, 7), 255)

            def step(i, ch0):
                chs = [ch0]
                for _ in range(_LANES - 1):
                    chs.append(chs[-1] + 1)
                xs = [plsc.load_gather(stage, [slot_v, c, src])
                      for c in chs]
                for c, x in zip(chs, xs):
                    plsc.store_scatter(outb, [zero16, c, dst], x)
                return chs[-1] + 1

            lax.fori_loop(0, _C // _LANES, step, zero16, unroll=False)
            return c2

        lax.fori_loop(0, gcnt, group, 0, unroll=False)

        @pl.when(vv + _NRING < _NV)
        def _():
            nxt = vv + _NRING
            start_stage(nxt // _NWPB, lax.rem(nxt, _NWPB), slot)

        @pl.when(w == _NWPB - 1)
        def _():
            pltpu.async_copy(outb.at[0], out.at[base_b + bb], psem)
        return carry

    lax.fori_loop(0, _NV, window, 0, unroll=False)
    wait_put()


_gather = functools.partial(
    pl.kernel,
    out_type=jax.ShapeDtypeStruct((_B, _C, _R), jnp.float32),
    mesh=plsc.VectorSubcoreMesh(core_axis_name="c", subcore_axis_name="s"),
    scratch_types=[
        pltpu.VMEM((1, _SCHED_ROWS, 128), jnp.int32),
        pltpu.VMEM((_NRING, _C, _WIN), jnp.float32),
        pltpu.VMEM((1, _C, _R), jnp.float32),
        pltpu.SemaphoreType.DMA((_NRING,)),
        pltpu.SemaphoreType.DMA,
    ],
    compiler_params=pltpu.CompilerParams(needs_layout_passes=False),
)(_body)


def _perm_constants():
    import contextlib
    try:
        cpu_mesh = jax.sharding.Mesh(np.array(jax.devices("cpu")[:1]), ("_",))
        ctx = jax.set_mesh(cpu_mesh)
    except Exception:
        ctx = contextlib.nullcontext()
    with ctx:
        keys = jax.random.split(jax.random.key(42), _B)
        fwd = jax.vmap(lambda k: jax.random.permutation(k, _T))(keys).astype(
            jnp.int64)
        bwd = jnp.argsort(fwd, axis=1)
    return np.asarray(fwd), np.asarray(bwd)


_FWD_NP, _BWD_NP = _perm_constants()
_CACHE = []


def kernel(patches):
    if not _CACHE:
        _CACHE.append(jnp.asarray(_build_schedule(_FWD_NP)))
    sched = _CACHE[0]
    out_t = _gather(jnp.swapaxes(patches, 1, 2), sched)
    return (jnp.swapaxes(out_t, 1, 2),
            jnp.asarray(_FWD_NP), jnp.asarray(_BWD_NP))


# speedup vs baseline: 9.3425x; 1.0908x over previous
import functools

import numpy as np

import jax
import jax.numpy as jnp
from jax import lax
from jax.experimental import pallas as pl
from jax.experimental.pallas import tpu as pltpu
from jax.experimental.pallas import tpu_sc as plsc

_RATIO = 0.75
_B, _T, _C = 256, 1024, 192
_R = int(_T * (1 - _RATIO))
_NC, _NS = 2, 16
_NW = _NC * _NS
_BPW = _B // _NW
_CW = 16
_NWPB = _C // _CW
_NV = _BPW * _NWPB
_NRING = 3
_LANES = 16
_NG = _R // _LANES
_SCHED_ROWS = _BPW * _NG * _LANES // 128


def _build_schedule(fwd_np):
    srcs = np.sort(fwd_np[:, :_R], axis=1)
    order = np.argsort(fwd_np[:, :_R], axis=1)
    sched = np.zeros((_NW, _BPW, _NG, _LANES), dtype=np.int32)
    for wid in range(_NW):
        for bb in range(_BPW):
            b = wid * _BPW + bb
            packed = (srcs[b] | (order[b] << 10)).astype(np.int32)
            sched[wid, bb] = packed.reshape(_NG, _LANES)
    return sched.reshape(_NW, _SCHED_ROWS, 128)


def _body(table, sched_hbm, out, sched_v, stage, outb, ssem, psem):
    wid = lax.axis_index("s") * _NC + lax.axis_index("c")
    base_b = wid * _BPW
    pltpu.sync_copy(sched_hbm.at[wid], sched_v.at[0])

    def start_stage(bb, cw, slot):
        src = table.at[base_b + bb, pl.ds(cw * _CW, _CW), :]
        return pltpu.async_copy(src, stage.at[slot], ssem.at[slot])

    for vv in range(_NRING):
        start_stage(vv // _NWPB, vv % _NWPB, vv % _NRING)

    def wait_stage(slot):
        pltpu.make_async_copy(
            table.at[0, pl.ds(0, _CW), :], stage.at[slot],
            ssem.at[slot]).wait()

    def wait_put():
        pltpu.make_async_copy(outb.at[0], out.at[0], psem).wait()

    zero16 = jnp.zeros((_LANES,), jnp.int32)

    def window(vv, carry):
        bb = vv // _NWPB
        cw = lax.rem(vv, _NWPB)
        slot = lax.rem(vv, _NRING)
        slot_v = zero16 + slot
        cbase = zero16 + cw * _CW
        wait_stage(slot)
        @pl.when(jnp.logical_and(cw == 0, vv > 0))
        def _():
            wait_put()

        def group(gg, c2):
            e = (bb * _NG + gg) * _LANES
            vg = sched_v[0, lax.div(e, 128), pl.ds(lax.rem(e, 128), _LANES)]
            src = lax.bitwise_and(vg, 1023)
            dst = lax.shift_right_logical(vg, 10)
            cl = [zero16]
            cg = [cbase]
            for _ in range(_CW - 1):
                cl.append(cl[-1] + 1)
                cg.append(cg[-1] + 1)
            xs = [plsc.load_gather(stage, [slot_v, c, src]) for c in cl]
            for c, x in zip(cg, xs):
                plsc.store_scatter(outb, [zero16, c, dst], x)
            return c2

        lax.fori_loop(0, _NG, group, 0, unroll=False)

        @pl.when(vv + _NRING < _NV)
        def _():
            nxt = vv + _NRING
            start_stage(nxt // _NWPB, lax.rem(nxt, _NWPB), slot)

        @pl.when(cw == _NWPB - 1)
        def _():
            pltpu.async_copy(outb.at[0], out.at[base_b + bb], psem)
        return carry

    lax.fori_loop(0, _NV, window, 0, unroll=False)
    wait_put()


_gather = functools.partial(
    pl.kernel,
    out_type=jax.ShapeDtypeStruct((_B, _C, _R), jnp.float32),
    mesh=plsc.VectorSubcoreMesh(core_axis_name="c", subcore_axis_name="s"),
    scratch_types=[
        pltpu.VMEM((1, _SCHED_ROWS, 128), jnp.int32),
        pltpu.VMEM((_NRING, _CW, _T), jnp.float32),
        pltpu.VMEM((1, _C, _R), jnp.float32),
        pltpu.SemaphoreType.DMA((_NRING,)),
        pltpu.SemaphoreType.DMA,
    ],
    compiler_params=pltpu.CompilerParams(needs_layout_passes=False),
)(_body)


def _perm_constants():
    import contextlib
    try:
        cpu_mesh = jax.sharding.Mesh(np.array(jax.devices("cpu")[:1]), ("_",))
        ctx = jax.set_mesh(cpu_mesh)
    except Exception:
        ctx = contextlib.nullcontext()
    with ctx:
        keys = jax.random.split(jax.random.key(42), _B)
        fwd = jax.vmap(lambda k: jax.random.permutation(k, _T))(keys).astype(
            jnp.int64)
        bwd = jnp.argsort(fwd, axis=1)
    return np.asarray(fwd), np.asarray(bwd)


_FWD_NP, _BWD_NP = _perm_constants()
_CACHE = []


def kernel(patches):
    if not _CACHE:
        _CACHE.append(jnp.asarray(_build_schedule(_FWD_NP)))
    sched = _CACHE[0]
    out_t = _gather(jnp.swapaxes(patches, 1, 2), sched)
    return (jnp.swapaxes(out_t, 1, 2),
            jnp.asarray(_FWD_NP), jnp.asarray(_BWD_NP))


# speedup vs baseline: 9.4588x; 1.0125x over previous
import functools

import numpy as np

import jax
import jax.numpy as jnp
from jax import lax
from jax.experimental import pallas as pl
from jax.experimental.pallas import tpu as pltpu
from jax.experimental.pallas import tpu_sc as plsc

_RATIO = 0.75
_B, _T, _C = 256, 1024, 192
_R = int(_T * (1 - _RATIO))
_NC, _NS = 2, 16
_NW = _NC * _NS
_BPW = _B // _NW
_CW = 16
_NWPB = _C // _CW
_NV = _BPW * _NWPB
_NRING = 4
_LANES = 16
_NG = _R // _LANES
_SCHED_ROWS = _BPW * _NG * _LANES // 128


def _build_schedule(fwd_np):
    srcs = np.sort(fwd_np[:, :_R], axis=1)
    order = np.argsort(fwd_np[:, :_R], axis=1)
    sched = np.zeros((_NW, _BPW, _NG, _LANES), dtype=np.int32)
    for wid in range(_NW):
        for bb in range(_BPW):
            b = wid * _BPW + bb
            packed = (srcs[b] | (order[b] << 10)).astype(np.int32)
            sched[wid, bb] = packed.reshape(_NG, _LANES)
    return sched.reshape(_NW, _SCHED_ROWS, 128)


def _body(table, sched_hbm, out, sched_v, stage, outb, ssem, psem):
    wid = lax.axis_index("s") * _NC + lax.axis_index("c")
    base_b = wid * _BPW
    pltpu.sync_copy(sched_hbm.at[wid], sched_v.at[0])

    def start_stage(bb, cw, slot):
        src = table.at[base_b + bb, pl.ds(cw * _CW, _CW), :]
        return pltpu.async_copy(src, stage.at[slot], ssem.at[slot])

    for vv in range(_NRING):
        start_stage(vv // _NWPB, vv % _NWPB, vv % _NRING)

    def wait_stage(slot):
        pltpu.make_async_copy(
            table.at[0, pl.ds(0, _CW), :], stage.at[slot],
            ssem.at[slot]).wait()

    def wait_put():
        pltpu.make_async_copy(outb.at[0], out.at[0], psem).wait()

    zero16 = jnp.zeros((_LANES,), jnp.int32)

    def window(vv, carry):
        bb = vv // _NWPB
        cw = lax.rem(vv, _NWPB)
        slot = lax.rem(vv, _NRING)
        slot_v = zero16 + slot
        cbase = zero16 + cw * _CW
        wait_stage(slot)
        @pl.when(jnp.logical_and(cw == 0, vv > 0))
        def _():
            wait_put()

        def group(gg, c2):
            e = (bb * _NG + gg) * _LANES
            vg = sched_v[0, lax.div(e, 128), pl.ds(lax.rem(e, 128), _LANES)]
            src = lax.bitwise_and(vg, 1023)
            dst = lax.shift_right_logical(vg, 10)
            cl = [zero16]
            cg = [cbase]
            for _ in range(_CW - 1):
                cl.append(cl[-1] + 1)
                cg.append(cg[-1] + 1)
            xs = [plsc.load_gather(stage, [slot_v, c, src]) for c in cl]
            for c, x in zip(cg, xs):
                plsc.store_scatter(outb, [zero16, c, dst], x)
            return c2

        lax.fori_loop(0, _NG, group, 0, unroll=False)

        @pl.when(vv + _NRING < _NV)
        def _():
            nxt = vv + _NRING
            start_stage(nxt // _NWPB, lax.rem(nxt, _NWPB), slot)

        @pl.when(cw == _NWPB - 1)
        def _():
            pltpu.async_copy(outb.at[0], out.at[base_b + bb], psem)
        return carry

    lax.fori_loop(0, _NV, window, 0, unroll=False)
    wait_put()


_gather = functools.partial(
    pl.kernel,
    out_type=jax.ShapeDtypeStruct((_B, _C, _R), jnp.float32),
    mesh=plsc.VectorSubcoreMesh(core_axis_name="c", subcore_axis_name="s"),
    scratch_types=[
        pltpu.VMEM((1, _SCHED_ROWS, 128), jnp.int32),
        pltpu.VMEM((_NRING, _CW, _T), jnp.float32),
        pltpu.VMEM((1, _C, _R), jnp.float32),
        pltpu.SemaphoreType.DMA((_NRING,)),
        pltpu.SemaphoreType.DMA,
    ],
    compiler_params=pltpu.CompilerParams(needs_layout_passes=False),
)(_body)


def _perm_constants():
    import contextlib
    try:
        cpu_mesh = jax.sharding.Mesh(np.array(jax.devices("cpu")[:1]), ("_",))
        ctx = jax.set_mesh(cpu_mesh)
    except Exception:
        ctx = contextlib.nullcontext()
    with ctx:
        keys = jax.random.split(jax.random.key(42), _B)
        fwd = jax.vmap(lambda k: jax.random.permutation(k, _T))(keys).astype(
            jnp.int64)
        bwd = jnp.argsort(fwd, axis=1)
    return np.asarray(fwd), np.asarray(bwd)


_FWD_NP, _BWD_NP = _perm_constants()
_CACHE = []


def kernel(patches):
    if not _CACHE:
        _CACHE.append(jnp.asarray(_build_schedule(_FWD_NP)))
    sched = _CACHE[0]
    out_t = _gather(jnp.swapaxes(patches, 1, 2), sched)
    return (jnp.swapaxes(out_t, 1, 2),
            jnp.asarray(_FWD_NP), jnp.asarray(_BWD_NP))


# speedup vs baseline: 9.9404x; 1.0509x over previous
import functools

import numpy as np

import jax
import jax.numpy as jnp
from jax import lax
from jax.experimental import pallas as pl
from jax.experimental.pallas import tpu as pltpu
from jax.experimental.pallas import tpu_sc as plsc

_RATIO = 0.75
_B, _T, _C = 256, 1024, 192
_R = int(_T * (1 - _RATIO))
_NC, _NS = 2, 16
_NW = _NC * _NS
_TCB = 64
_SCB = _B - _TCB
_BPW = _SCB // _NW
_CW = 16
_NWPB = _C // _CW
_NV = _BPW * _NWPB
_NRING = 4
_LANES = 16
_NG = _R // _LANES
_SCHED_ROWS = _BPW * _NG * _LANES // 128


def _build_schedule(fwd_np):
    srcs = np.sort(fwd_np[:, :_R], axis=1)
    order = np.argsort(fwd_np[:, :_R], axis=1)
    sched = np.zeros((_NW, _BPW, _NG, _LANES), dtype=np.int32)
    for wid in range(_NW):
        for bb in range(_BPW):
            b = wid * _BPW + bb
            packed = (srcs[b] | (order[b] << 10)).astype(np.int32)
            sched[wid, bb] = packed.reshape(_NG, _LANES)
    return sched.reshape(_NW, _SCHED_ROWS, 128)


def _body(table, sched_hbm, out, sched_v, stage, outb, ssem, psem):
    wid = lax.axis_index("s") * _NC + lax.axis_index("c")
    base_b = wid * _BPW
    pltpu.sync_copy(sched_hbm.at[wid], sched_v.at[0])

    def start_stage(bb, cw, slot):
        src = table.at[base_b + bb, pl.ds(cw * _CW, _CW), :]
        return pltpu.async_copy(src, stage.at[slot], ssem.at[slot])

    for vv in range(_NRING):
        start_stage(vv // _NWPB, vv % _NWPB, vv % _NRING)

    def wait_stage(slot):
        pltpu.make_async_copy(
            table.at[0, pl.ds(0, _CW), :], stage.at[slot],
            ssem.at[slot]).wait()

    def wait_put():
        pltpu.make_async_copy(outb.at[0], out.at[0], psem).wait()

    zero16 = jnp.zeros((_LANES,), jnp.int32)

    def window(vv, carry):
        bb = vv // _NWPB
        cw = lax.rem(vv, _NWPB)
        slot = lax.rem(vv, _NRING)
        slot_v = zero16 + slot
        cbase = zero16 + cw * _CW
        wait_stage(slot)
        @pl.when(jnp.logical_and(cw == 0, vv > 0))
        def _():
            wait_put()

        def group(gg, c2):
            e = (bb * _NG + gg) * _LANES
            vg = sched_v[0, lax.div(e, 128), pl.ds(lax.rem(e, 128), _LANES)]
            src = lax.bitwise_and(vg, 1023)
            dst = lax.shift_right_logical(vg, 10)
            cl = [zero16]
            cg = [cbase]
            for _ in range(_CW - 1):
                cl.append(cl[-1] + 1)
                cg.append(cg[-1] + 1)
            xs = [plsc.load_gather(stage, [slot_v, c, src]) for c in cl]
            for c, x in zip(cg, xs):
                plsc.store_scatter(outb, [zero16, c, dst], x)
            return c2

        lax.fori_loop(0, _NG, group, 0, unroll=False)

        @pl.when(vv + _NRING < _NV)
        def _():
            nxt = vv + _NRING
            start_stage(nxt // _NWPB, lax.rem(nxt, _NWPB), slot)

        @pl.when(cw == _NWPB - 1)
        def _():
            pltpu.async_copy(outb.at[0], out.at[base_b + bb], psem)
        return carry

    lax.fori_loop(0, _NV, window, 0, unroll=False)
    wait_put()


_gather = functools.partial(
    pl.kernel,
    out_type=jax.ShapeDtypeStruct((_B, _C, _R), jnp.float32),
    mesh=plsc.VectorSubcoreMesh(core_axis_name="c", subcore_axis_name="s"),
    scratch_types=[
        pltpu.VMEM((1, _SCHED_ROWS, 128), jnp.int32),
        pltpu.VMEM((_NRING, _CW, _T), jnp.float32),
        pltpu.VMEM((1, _C, _R), jnp.float32),
        pltpu.SemaphoreType.DMA((_NRING,)),
        pltpu.SemaphoreType.DMA,
    ],
    compiler_params=pltpu.CompilerParams(needs_layout_passes=False),
)(_body)


def _tc_body(idx_ref, in_ref, o_ref):
    idx = idx_ref[0, 0, :]
    ts = lax.broadcasted_iota(jnp.int32, (_T, _R), 0)
    onehot = (ts == idx[None, :]).astype(jnp.float32)
    o_ref[0] = jnp.dot(in_ref[0], onehot,
                       preferred_element_type=jnp.float32)


_tc_gather = pl.pallas_call(
    _tc_body,
    grid=(_TCB,),
    in_specs=[
        pl.BlockSpec((1, 1, _R), lambda i: (i, 0, 0)),
        pl.BlockSpec((1, _C, _T), lambda i: (_SCB + i, 0, 0)),
    ],
    out_specs=pl.BlockSpec((1, _C, _R), lambda i: (i, 0, 0)),
    out_shape=jax.ShapeDtypeStruct((_TCB, _C, _R), jnp.float32),
)


def _perm_constants():
    import contextlib
    try:
        cpu_mesh = jax.sharding.Mesh(np.array(jax.devices("cpu")[:1]), ("_",))
        ctx = jax.set_mesh(cpu_mesh)
    except Exception:
        ctx = contextlib.nullcontext()
    with ctx:
        keys = jax.random.split(jax.random.key(42), _B)
        fwd = jax.vmap(lambda k: jax.random.permutation(k, _T))(keys).astype(
            jnp.int64)
        bwd = jnp.argsort(fwd, axis=1)
    return np.asarray(fwd), np.asarray(bwd)


_FWD_NP, _BWD_NP = _perm_constants()
_CACHE = []


def kernel(patches):
    if not _CACHE:
        tc_idx = _FWD_NP[_SCB:, :_R].reshape(_TCB, 1, _R).astype(np.int32)
        _CACHE.append((jnp.asarray(_build_schedule(_FWD_NP)),
                       jnp.asarray(tc_idx)))
    sched, tc_idx = _CACHE[0]
    table = jnp.swapaxes(patches, 1, 2)
    sc_out = _gather(table, sched)
    tc_out = _tc_gather(tc_idx, table)
    out_t = lax.dynamic_update_slice(sc_out, tc_out, (_SCB, 0, 0))
    return (jnp.swapaxes(out_t, 1, 2),
            jnp.asarray(_FWD_NP), jnp.asarray(_BWD_NP))
